# baseline (speedup 1.0000x reference)
import functools

import jax
import jax.numpy as jnp
from jax import lax
from jax.experimental import pallas as pl
from jax.experimental.pallas import tpu as pltpu
from jax.experimental.pallas import tpu_sc as plsc

T = 8
SIZE = 1024
NBUF = 2
CHUNKS = 4


def _frame_gather(noise, idx1):
    info = plsc.get_sparse_core_info()
    nc, ns = info.num_cores, info.num_subcores
    nw = nc * ns
    slab = SIZE // nw
    crows = slab // CHUNKS
    mesh = plsc.VectorSubcoreMesh(core_axis_name="c", subcore_axis_name="s")

    @functools.partial(
        pl.kernel,
        mesh=mesh,
        out_type=jax.ShapeDtypeStruct((1, 1, SIZE, SIZE), jnp.float32),
        scratch_types=[
            pltpu.VMEM((16,), jnp.int32),
            pltpu.VMEM((slab, SIZE), jnp.float32),
        ],
    )
    def body(noise_hbm, idx_hbm, out_hbm, idx_vmem, buf_vmem):
        wid = lax.axis_index("s") * nc + lax.axis_index("c")
        pltpu.sync_copy(idx_hbm, idx_vmem.at[pl.ds(0, 1)])
        i = idx_vmem[...][0]
        base = wid * slab
        pltpu.sync_copy(noise_hbm.at[i, 0, pl.ds(base, slab), :], buf_vmem)
        pltpu.sync_copy(buf_vmem, out_hbm.at[0, 0, pl.ds(base, slab), :])

    return body(noise, idx1)


def kernel(noise, index):
    idx1 = jnp.asarray(index, jnp.int32).reshape(1)
    return _frame_gather(noise, idx1)


# speedup vs baseline: 1.0061x; 1.0061x over previous
import functools

import jax
import jax.numpy as jnp
from jax import lax
from jax.experimental import pallas as pl
from jax.experimental.pallas import tpu as pltpu
from jax.experimental.pallas import tpu_sc as plsc

T = 8
SIZE = 1024


def _frame_gather(noise, idx1):
    info = plsc.get_sparse_core_info()
    nc, ns = info.num_cores, info.num_subcores
    nw = nc * ns
    slab = SIZE // nw
    mesh = plsc.VectorSubcoreMesh(core_axis_name="c", subcore_axis_name="s")

    @functools.partial(
        pl.kernel,
        mesh=mesh,
        out_type=jax.ShapeDtypeStruct((1, 1, SIZE, SIZE), jnp.float32),
        scratch_types=[
            pltpu.VMEM((16,), jnp.int32),
            pltpu.VMEM((slab, SIZE), jnp.float32),
        ],
    )
    def body(noise_hbm, idx_hbm, out_hbm, idx_vmem, buf_vmem):
        wid = lax.axis_index("s") * nc + lax.axis_index("c")
        pltpu.sync_copy(idx_hbm, idx_vmem.at[pl.ds(0, 1)])
        i = idx_vmem[...][0]
        base = wid * slab
        pltpu.sync_copy(noise_hbm.at[i, 0, pl.ds(base, slab), :], buf_vmem)
        pltpu.sync_copy(buf_vmem, out_hbm.at[0, 0, pl.ds(base, slab), :])

    return body(noise, idx1)


def kernel(noise, index):
    idx1 = jnp.asarray(index, jnp.int32).reshape(1)
    return _frame_gather(noise, idx1)
